# baseline (speedup 1.0000x reference)
import functools

import jax
import jax.numpy as jnp
from jax import lax
from jax.experimental import pallas as pl
from jax.experimental.pallas import tpu as pltpu
from jax.experimental.pallas import tpu_sc as plsc

_B = 16384
_D = 512
_OUT_LO = 120
_OUT_HI = 136
_NC = 2
_NS = 16
_NW = _NC * _NS
_RPW = _B // _NW
_EPW = _RPW * _D
_NIDX = 2 * _RPW
_IROWS = _NIDX // 128


def _sc_body(x_hbm, out_hbm, idx, vals, sem_cp, sem_sc):
    wid = lax.axis_index("s") * _NC + lax.axis_index("c")
    base = wid * _EPW
    lane = lax.iota(jnp.int32, 16)
    two = jnp.full((16,), 2.0, jnp.float32)
    cp = pltpu.async_copy(
        x_hbm.at[pl.ds(base, _EPW)], out_hbm.at[pl.ds(base, _EPW)], sem_cp)
    for j in range(_IROWS):
        for k in range(8):
            g0 = j * 128 + k * 16
            col = _OUT_LO if g0 < _RPW * 1 else _OUT_HI
            r0 = g0 % _RPW
            idx[j, pl.ds(k * 16, 16)] = base + (r0 + lane) * _D + col
        vals[j, pl.ds(0, 16)] = two
        vals[j, pl.ds(16, 16)] = two
        vals[j, pl.ds(32, 16)] = two
        vals[j, pl.ds(48, 16)] = two
        vals[j, pl.ds(64, 16)] = two
        vals[j, pl.ds(80, 16)] = two
        vals[j, pl.ds(96, 16)] = two
        vals[j, pl.ds(112, 16)] = two
    cp.wait()
    scs = []
    for j in range(_IROWS):
        scs.append(pltpu.async_copy(
            vals.at[j], out_hbm.at[idx.at[j]], sem_sc))
    for s in scs:
        s.wait()


_sc_call = functools.partial(
    pl.kernel,
    out_type=jax.ShapeDtypeStruct((_B * _D,), jnp.float32),
    mesh=plsc.VectorSubcoreMesh(core_axis_name="c", subcore_axis_name="s"),
    scratch_types=[
        pltpu.VMEM((_IROWS, 128), jnp.int32),
        pltpu.VMEM((_IROWS, 128), jnp.float32),
        pltpu.SemaphoreType.DMA,
        pltpu.SemaphoreType.DMA,
    ],
    compiler_params=pltpu.CompilerParams(
        needs_layout_passes=False, use_tc_tiling_on_sc=False),
)(_sc_body)


def kernel(x_bd):
    out = _sc_call(x_bd.reshape(_B * _D))
    return out.reshape(_B, 1, _D)


# speedup vs baseline: 24.1018x; 24.1018x over previous
import functools

import jax
import jax.numpy as jnp
from jax import lax
from jax.experimental import pallas as pl
from jax.experimental.pallas import tpu as pltpu
from jax.experimental.pallas import tpu_sc as plsc

_B = 16384
_D = 512
_OUT_LO = 120
_OUT_HI = 136
_NC = 2
_NS = 16
_NW = _NC * _NS
_RPW = _B // _NW
_C = 64
_NCHUNK = _RPW // _C


def _overwrite(buf, lane, two, col_lo, col_hi):
    for j in range(_C // 16):
        rows = lane + (j * 16)
        plsc.store_scatter(buf, [rows, col_lo], two)
        plsc.store_scatter(buf, [rows, col_hi], two)


def _sc_body(x_hbm, out_hbm, buf0, buf1, si0, si1, so0, so1):
    wid = lax.axis_index("s") * _NC + lax.axis_index("c")
    base = wid * _RPW
    lane = lax.iota(jnp.int32, 16)
    two = jnp.full((16,), 2.0, jnp.float32)
    col_lo = jnp.full((16,), _OUT_LO, jnp.int32)
    col_hi = jnp.full((16,), _OUT_HI, jnp.int32)
    bufs = (buf0, buf1)
    sin = (si0, si1)
    sout = (so0, so1)
    in_cp = [None, None]
    out_cp = [None, None]
    in_cp[0] = pltpu.async_copy(x_hbm.at[pl.ds(base, _C)], bufs[0], sin[0])
    for i in range(_NCHUNK):
        b = i % 2
        nb = 1 - b
        if i + 1 < _NCHUNK:
            if out_cp[nb] is not None:
                out_cp[nb].wait()
            in_cp[nb] = pltpu.async_copy(
                x_hbm.at[pl.ds(base + (i + 1) * _C, _C)], bufs[nb], sin[nb])
        in_cp[b].wait()
        _overwrite(bufs[b], lane, two, col_lo, col_hi)
        out_cp[b] = pltpu.async_copy(
            bufs[b], out_hbm.at[pl.ds(base + i * _C, _C)], sout[b])
    out_cp[(_NCHUNK - 2) % 2].wait()
    out_cp[(_NCHUNK - 1) % 2].wait()


_sc_call = functools.partial(
    pl.kernel,
    out_type=jax.ShapeDtypeStruct((_B, _D), jnp.float32),
    mesh=plsc.VectorSubcoreMesh(core_axis_name="c", subcore_axis_name="s"),
    scratch_types=[
        pltpu.VMEM((_C, _D), jnp.float32),
        pltpu.VMEM((_C, _D), jnp.float32),
        pltpu.SemaphoreType.DMA,
        pltpu.SemaphoreType.DMA,
        pltpu.SemaphoreType.DMA,
        pltpu.SemaphoreType.DMA,
    ],
    compiler_params=pltpu.CompilerParams(
        needs_layout_passes=False, use_tc_tiling_on_sc=False),
)(_sc_body)


def kernel(x_bd):
    out = _sc_call(x_bd.reshape(_B, _D))
    return out.reshape(_B, 1, _D)


# speedup vs baseline: 24.5371x; 1.0181x over previous
import functools

import jax
import jax.numpy as jnp
from jax import lax
from jax.experimental import pallas as pl
from jax.experimental.pallas import tpu as pltpu
from jax.experimental.pallas import tpu_sc as plsc

_B = 16384
_D = 512
_OUT_LO = 120
_OUT_HI = 136
_NC = 2
_NS = 16
_NW = _NC * _NS
_RPW = _B // _NW
_C = 64
_NCHUNK = _RPW // _C


def _overwrite(buf, lane, two, col_lo, col_hi):
    for j in range(_C // 16):
        rows = lane + (j * 16)
        plsc.store_scatter(buf, [rows, col_lo], two)
        plsc.store_scatter(buf, [rows, col_hi], two)


_NBUF = 3


def _sc_body(x_hbm, out_hbm, *scratch):
    bufs = scratch[:_NBUF]
    sin = scratch[_NBUF:2 * _NBUF]
    sout = scratch[2 * _NBUF:3 * _NBUF]
    wid = lax.axis_index("s") * _NC + lax.axis_index("c")
    base = wid * _RPW
    lane = lax.iota(jnp.int32, 16)
    two = jnp.full((16,), 2.0, jnp.float32)
    col_lo = jnp.full((16,), _OUT_LO, jnp.int32)
    col_hi = jnp.full((16,), _OUT_HI, jnp.int32)
    in_cp = [None] * _NBUF
    out_cp = [None] * _NBUF
    for i in range(min(_NBUF, _NCHUNK)):
        in_cp[i] = pltpu.async_copy(
            x_hbm.at[pl.ds(base + i * _C, _C)], bufs[i], sin[i])
    for i in range(_NCHUNK):
        b = i % _NBUF
        in_cp[b].wait()
        _overwrite(bufs[b], lane, two, col_lo, col_hi)
        out_cp[b] = pltpu.async_copy(
            bufs[b], out_hbm.at[pl.ds(base + i * _C, _C)], sout[b])
        nxt = i + _NBUF
        if nxt < _NCHUNK:
            out_cp[b].wait()
            in_cp[b] = pltpu.async_copy(
                x_hbm.at[pl.ds(base + nxt * _C, _C)], bufs[b], sin[b])
    for i in range(max(0, _NCHUNK - _NBUF), _NCHUNK):
        out_cp[i % _NBUF].wait()


_sc_call = functools.partial(
    pl.kernel,
    out_type=jax.ShapeDtypeStruct((_B, _D), jnp.float32),
    mesh=plsc.VectorSubcoreMesh(core_axis_name="c", subcore_axis_name="s"),
    scratch_types=(
        [pltpu.VMEM((_C, _D), jnp.float32)] * _NBUF
        + [pltpu.SemaphoreType.DMA] * (2 * _NBUF)
    ),
    compiler_params=pltpu.CompilerParams(
        needs_layout_passes=False, use_tc_tiling_on_sc=False),
)(_sc_body)


def kernel(x_bd):
    out = _sc_call(x_bd.reshape(_B, _D))
    return out.reshape(_B, 1, _D)
